# baseline (speedup 1.0000x reference)
import functools

import jax
import jax.numpy as jnp
from jax import lax
from jax.experimental import pallas as pl
from jax.experimental.pallas import tpu as pltpu
from jax.experimental.pallas import tpu_sc as plsc

M = 16384
K = 256
N = 8192
BM = 256
COMMITMENT_COST = 0.25

_NC = 2
_NS = 16
_NW = _NC * _NS
_CHUNK = 128
_TOK_PER_W = M // _NW
_NCHUNK = _TOK_PER_W // _CHUNK


def _vq_block(x_ref, w_ref, enc_ref, idx_ref, dmin_ref, wn_scr):
    i = pl.program_id(0)
    x = x_ref[...]
    w = w_ref[...]

    @pl.when(i == 0)
    def _():
        wsq = jnp.sum(w ** 2, axis=1, keepdims=True)
        wn_scr[...] = jnp.swapaxes(wsq, 0, 1)

    rev = (N - lax.broadcasted_iota(jnp.int32, (1, N), 1)).astype(jnp.float32)
    rn = jnp.sum(x ** 2, axis=1, keepdims=True)
    x2 = x + x
    mm2 = jax.lax.dot_general(
        x2, w, (((1,), (1,)), ((), ())),
        preferred_element_type=jnp.float32)
    d = (rn + wn_scr[...]) - mm2
    m = jnp.min(d, axis=1, keepdims=True)
    v = jnp.where(d == m, rev, 0.0)
    vmax = jnp.max(v, axis=1, keepdims=True)
    enc_ref[...] = (v == vmax).astype(jnp.float32)
    vmax_t = jnp.swapaxes(vmax, 0, 1)
    idx_ref[...] = (float(N) - vmax_t).astype(jnp.int32).reshape(1, 1, BM)
    dmin_ref[...] = jnp.swapaxes(m, 0, 1).reshape(1, 1, BM)


def _sc_gather_body(w_hbm, idx_hbm, out_hbm, idx_a, idx_b, rows_a, rows_b,
                    sem_a, sem_b):
    wid = lax.axis_index("s") * _NC + lax.axis_index("c")
    base = wid * _TOK_PER_W
    idxs, rows, sems = (idx_a, idx_b), (rows_a, rows_b), (sem_a, sem_b)
    pltpu.sync_copy(idx_hbm.at[pl.ds(base, _CHUNK)], idx_a)
    copies = [pltpu.async_copy(w_hbm.at[idx_a], rows_a, sem_a)]
    for c in range(_NCHUNK):
        nb = (c + 1) % 2
        if c + 1 < _NCHUNK:
            off_n = base + (c + 1) * _CHUNK
            pltpu.sync_copy(idx_hbm.at[pl.ds(off_n, _CHUNK)], idxs[nb])
            copies.append(
                pltpu.async_copy(w_hbm.at[idxs[nb]], rows[nb], sems[nb]))
        copies[c].wait()
        pltpu.sync_copy(rows[c % 2], out_hbm.at[pl.ds(base + c * _CHUNK, _CHUNK)])


def kernel(c_input, weight):
    enc, idx, dmin = pl.pallas_call(
        _vq_block,
        grid=(M // BM,),
        in_specs=[
            pl.BlockSpec((BM, K), lambda i: (i, 0)),
            pl.BlockSpec((N, K), lambda i: (0, 0)),
        ],
        out_specs=[
            pl.BlockSpec((BM, N), lambda i: (i, 0)),
            pl.BlockSpec((1, 1, BM), lambda i: (i, 0, 0)),
            pl.BlockSpec((1, 1, BM), lambda i: (i, 0, 0)),
        ],
        out_shape=[
            jax.ShapeDtypeStruct((M, N), jnp.float32),
            jax.ShapeDtypeStruct((M // BM, 1, BM), jnp.int32),
            jax.ShapeDtypeStruct((M // BM, 1, BM), jnp.float32),
        ],
        scratch_shapes=[pltpu.VMEM((1, N), jnp.float32)],
    )(c_input, weight)

    mesh = plsc.VectorSubcoreMesh(core_axis_name="c", subcore_axis_name="s")
    gather = functools.partial(
        pl.kernel,
        out_type=jax.ShapeDtypeStruct((M, K), jnp.float32),
        mesh=mesh,
        scratch_types=[
            pltpu.VMEM((_CHUNK,), jnp.int32),
            pltpu.VMEM((_CHUNK,), jnp.int32),
            pltpu.VMEM((_CHUNK, K), jnp.float32),
            pltpu.VMEM((_CHUNK, K), jnp.float32),
            pltpu.SemaphoreType.DMA,
            pltpu.SemaphoreType.DMA,
        ],
    )(_sc_gather_body)
    quantized = gather(weight, idx.reshape(M))
    s = jnp.sum(dmin) / (M * K)
    loss = s + COMMITMENT_COST * s
    return (loss, quantized, enc)


# speedup vs baseline: 1.0923x; 1.0923x over previous
import functools

import jax
import jax.numpy as jnp
from jax import lax
from jax.experimental import pallas as pl
from jax.experimental.pallas import tpu as pltpu
from jax.experimental.pallas import tpu_sc as plsc

M = 16384
K = 256
N = 8192
BM = 256
COMMITMENT_COST = 0.25

_NC = 2
_NS = 16
_NW = _NC * _NS
_CHUNK = 128
_TOK_PER_W = M // _NW
_NCHUNK = _TOK_PER_W // _CHUNK


def _vq_block(x_ref, w_ref, wn_ref, enc_ref, idx_ref, dmin_ref):
    x = x_ref[...]
    w = w_ref[...]
    rev = (N - lax.broadcasted_iota(jnp.int32, (1, N), 1)).astype(jnp.float32)
    rn = jnp.sum(x ** 2, axis=1, keepdims=True)
    x2 = x + x
    mm2 = jax.lax.dot_general(
        x2, w, (((1,), (1,)), ((), ())),
        preferred_element_type=jnp.float32)
    d = (rn + wn_ref[...]) - mm2
    m = jnp.min(d, axis=1, keepdims=True)
    v = jnp.where(d == m, rev, 0.0)
    vmax = jnp.max(v, axis=1, keepdims=True)
    enc_ref[...] = (v == vmax).astype(jnp.float32)
    vmax_t = jnp.swapaxes(vmax, 0, 1)
    idx_ref[...] = (float(N) - vmax_t).astype(jnp.int32).reshape(1, 1, BM)
    dmin_ref[...] = jnp.swapaxes(m, 0, 1).reshape(1, 1, BM)


def _sc_gather_body(w_hbm, idx_hbm, out_hbm, idx_a, idx_b, rows_a, rows_b,
                    sem_a, sem_b):
    wid = lax.axis_index("s") * _NC + lax.axis_index("c")
    base = wid * _TOK_PER_W
    idxs, rows, sems = (idx_a, idx_b), (rows_a, rows_b), (sem_a, sem_b)
    pltpu.sync_copy(idx_hbm.at[pl.ds(base, _CHUNK)], idx_a)
    copies = [pltpu.async_copy(w_hbm.at[idx_a], rows_a, sem_a)]
    for c in range(_NCHUNK):
        nb = (c + 1) % 2
        if c + 1 < _NCHUNK:
            off_n = base + (c + 1) * _CHUNK
            pltpu.sync_copy(idx_hbm.at[pl.ds(off_n, _CHUNK)], idxs[nb])
            copies.append(
                pltpu.async_copy(w_hbm.at[idxs[nb]], rows[nb], sems[nb]))
        copies[c].wait()
        pltpu.sync_copy(rows[c % 2], out_hbm.at[pl.ds(base + c * _CHUNK, _CHUNK)])


def kernel(c_input, weight):
    wn = jnp.sum(weight ** 2, axis=1)[None, :]
    enc, idx, dmin = pl.pallas_call(
        _vq_block,
        grid=(M // BM,),
        in_specs=[
            pl.BlockSpec((BM, K), lambda i: (i, 0)),
            pl.BlockSpec((N, K), lambda i: (0, 0)),
            pl.BlockSpec((1, N), lambda i: (0, 0)),
        ],
        out_specs=[
            pl.BlockSpec((BM, N), lambda i: (i, 0)),
            pl.BlockSpec((1, 1, BM), lambda i: (i, 0, 0)),
            pl.BlockSpec((1, 1, BM), lambda i: (i, 0, 0)),
        ],
        out_shape=[
            jax.ShapeDtypeStruct((M, N), jnp.float32),
            jax.ShapeDtypeStruct((M // BM, 1, BM), jnp.int32),
            jax.ShapeDtypeStruct((M // BM, 1, BM), jnp.float32),
        ],
    )(c_input, weight, wn)

    mesh = plsc.VectorSubcoreMesh(core_axis_name="c", subcore_axis_name="s")
    gather = functools.partial(
        pl.kernel,
        out_type=jax.ShapeDtypeStruct((M, K), jnp.float32),
        mesh=mesh,
        scratch_types=[
            pltpu.VMEM((_CHUNK,), jnp.int32),
            pltpu.VMEM((_CHUNK,), jnp.int32),
            pltpu.VMEM((_CHUNK, K), jnp.float32),
            pltpu.VMEM((_CHUNK, K), jnp.float32),
            pltpu.SemaphoreType.DMA,
            pltpu.SemaphoreType.DMA,
        ],
    )(_sc_gather_body)
    quantized = gather(weight, idx.reshape(M))
    s = jnp.sum(dmin) / (M * K)
    loss = s + COMMITMENT_COST * s
    return (loss, quantized, enc)
